# baseline (speedup 1.0000x reference)
import jax
import jax.numpy as jnp
from jax import lax
from jax.experimental import pallas as pl
from jax.experimental.pallas import tpu as pltpu
from jax.experimental.pallas import tpu_sc as plsc

_H = 32
_W = 32
_D = 256
_B = 8
_C = 2 * _D
_NC = 2
_NS = 16
_NW = _NC * _NS
_CH = _C // _NW
_L = 16
_HW = _H * _W
_TILE = _CH * _HW
_BSTRIDE = _C * _HW


def _pos_body(row_hbm, col_hbm, out_hbm, row_v, col_v, exp_v, sem):
    wid = lax.axis_index("s") * _NC + lax.axis_index("c")
    pltpu.sync_copy(row_hbm.at[pl.ds(0, _H * _D)], row_v)
    pltpu.sync_copy(col_hbm.at[pl.ds(0, _W * _D)], col_v)
    lanes = lax.iota(jnp.int32, _L)
    c0 = wid * _CH

    @pl.when(wid < _NW // 2)
    def _col_channels():
        for ch in range(_CH):
            cc = jnp.full((_L,), c0 + ch, jnp.int32)
            lo = plsc.load_gather(col_v, [lanes * _D + cc])
            hi = plsc.load_gather(col_v, [(lanes + _L) * _D + cc])
            for i in range(_H):
                base = ch * _HW + i * _W
                exp_v[pl.ds(base, _L)] = lo
                exp_v[pl.ds(base + _L, _L)] = hi

    @pl.when(wid >= _NW // 2)
    def _row_channels():
        for ch in range(_CH):
            cc = jnp.full((_L,), (c0 - _D) + ch, jnp.int32)
            for i in range(_H):
                v = plsc.load_gather(row_v, [cc + i * _D])
                base = ch * _HW + i * _W
                exp_v[pl.ds(base, _L)] = v
                exp_v[pl.ds(base + _L, _L)] = v

    out0 = c0 * _HW
    copies = [
        pltpu.async_copy(exp_v, out_hbm.at[pl.ds(b * _BSTRIDE + out0, _TILE)],
                         sem)
        for b in range(_B)
    ]
    for cp in copies:
        cp.wait()


@jax.jit
def _pos_embed(row_flat, col_flat):
    mesh = plsc.VectorSubcoreMesh(core_axis_name="c", subcore_axis_name="s")
    out = pl.kernel(
        _pos_body,
        out_type=jax.ShapeDtypeStruct((_B * _C * _HW,), jnp.float32),
        mesh=mesh,
        scratch_types=[
            pltpu.VMEM((_H * _D,), jnp.float32),
            pltpu.VMEM((_W * _D,), jnp.float32),
            pltpu.VMEM((_TILE,), jnp.float32),
            pltpu.SemaphoreType.DMA,
        ],
        compiler_params=pltpu.CompilerParams(needs_layout_passes=False),
    )(row_flat, col_flat)
    return out.reshape(_B, _C, _H, _W)


def kernel(x, row_embed, col_embed):
    assert x.shape[0] == _B and x.shape[-2:] == (_H, _W)
    return _pos_embed(row_embed.reshape(-1), col_embed.reshape(-1))


# speedup vs baseline: 1.1084x; 1.1084x over previous
import jax
import jax.numpy as jnp
from jax import lax
from jax.experimental import pallas as pl
from jax.experimental.pallas import tpu as pltpu
from jax.experimental.pallas import tpu_sc as plsc

_H = 32
_W = 32
_D = 256
_B = 8
_C = 2 * _D
_NC = 2
_NS = 16
_NW = _NC * _NS
_CH = _C // _NW
_L = 16


def _pos_body(row_hbm, col_hbm, out_hbm, row_v, col_v, exp_v, sem):
    wid = lax.axis_index("s") * _NC + lax.axis_index("c")
    pltpu.sync_copy(row_hbm.at[pl.ds(0, _H)], row_v)
    pltpu.sync_copy(col_hbm.at[pl.ds(0, _W)], col_v)
    lanes = lax.iota(jnp.int32, _L)
    c0 = wid * _CH

    @pl.when(wid < _NW // 2)
    def _col_channels():
        for ch in range(_CH):
            cc = jnp.full((_L,), c0 + ch, jnp.int32)
            lo = plsc.load_gather(col_v, [lanes, cc])
            hi = plsc.load_gather(col_v, [lanes + _L, cc])
            for i in range(_H):
                exp_v[ch, i, pl.ds(0, _L)] = lo
                exp_v[ch, i, pl.ds(_L, _L)] = hi

    @pl.when(wid >= _NW // 2)
    def _row_channels():
        for ch in range(_CH):
            cc = jnp.full((_L,), (c0 - _D) + ch, jnp.int32)
            for i in range(_H):
                ii = jnp.full((_L,), i, jnp.int32)
                v = plsc.load_gather(row_v, [ii, cc])
                exp_v[ch, i, pl.ds(0, _L)] = v
                exp_v[ch, i, pl.ds(_L, _L)] = v

    copies = [pltpu.async_copy(exp_v, out_hbm.at[b, pl.ds(c0, _CH)], sem)
              for b in range(_B)]
    for cp in copies:
        cp.wait()


@jax.jit
def _pos_embed(row_embed, col_embed):
    mesh = plsc.VectorSubcoreMesh(core_axis_name="c", subcore_axis_name="s")
    return pl.kernel(
        _pos_body,
        out_type=jax.ShapeDtypeStruct((_B, _C, _H, _W), jnp.float32),
        mesh=mesh,
        scratch_types=[
            pltpu.VMEM((_H, _D), jnp.float32),
            pltpu.VMEM((_W, _D), jnp.float32),
            pltpu.VMEM((_CH, _H, _W), jnp.float32),
            pltpu.SemaphoreType.DMA,
        ],
        compiler_params=pltpu.CompilerParams(needs_layout_passes=False),
    )(row_embed, col_embed)


def kernel(x, row_embed, col_embed):
    assert x.shape[0] == _B and x.shape[-2:] == (_H, _W)
    return _pos_embed(row_embed, col_embed)


# speedup vs baseline: 3.2595x; 2.9408x over previous
import jax
import jax.numpy as jnp
from jax import lax
from jax.experimental import pallas as pl
from jax.experimental.pallas import tpu as pltpu
from jax.experimental.pallas import tpu_sc as plsc

_H = 32
_W = 32
_D = 256
_B = 8
_C = 2 * _D
_NC = 2
_NS = 16
_NW = _NC * _NS
_L = 16


def _pos_body(row_hbm, col_hbm, out_hbm, row_v, slab_v, sem):
    i = lax.axis_index("s") * _NC + lax.axis_index("c")
    pltpu.sync_copy(col_hbm.at[pl.ds(0, _W)], slab_v.at[:, pl.ds(0, _D)])
    pltpu.sync_copy(row_hbm.at[pl.ds(0, _H)], row_v)
    lanes = lax.iota(jnp.int32, _L)
    ii = jnp.full((_L,), i, jnp.int32)
    for k in range(_D // _L):
        v = plsc.load_gather(row_v, [ii, lanes + k * _L])
        for j in range(_W):
            slab_v[j, pl.ds(_D + k * _L, _L)] = v
    copies = [pltpu.async_copy(slab_v, out_hbm.at[b, i], sem)
              for b in range(_B)]
    for cp in copies:
        cp.wait()


@jax.jit
def _pos_embed(row_embed, col_embed):
    mesh = plsc.VectorSubcoreMesh(core_axis_name="c", subcore_axis_name="s")
    out = pl.kernel(
        _pos_body,
        out_type=jax.ShapeDtypeStruct((_B, _H, _W, _C), jnp.float32),
        mesh=mesh,
        scratch_types=[
            pltpu.VMEM((_H, _D), jnp.float32),
            pltpu.VMEM((_W, _C), jnp.float32),
            pltpu.SemaphoreType.DMA,
        ],
        compiler_params=pltpu.CompilerParams(needs_layout_passes=False),
    )(row_embed, col_embed)
    return jnp.transpose(out, (0, 3, 1, 2))


def kernel(x, row_embed, col_embed):
    assert x.shape[0] == _B and x.shape[-2:] == (_H, _W)
    return _pos_embed(row_embed, col_embed)


# speedup vs baseline: 3.2605x; 1.0003x over previous
import jax
import jax.numpy as jnp
from jax import lax
from jax.experimental import pallas as pl
from jax.experimental.pallas import tpu as pltpu
from jax.experimental.pallas import tpu_sc as plsc

_H = 32
_W = 32
_D = 256
_B = 8
_C = 2 * _D
_NC = 2
_NS = 16
_NW = _NC * _NS
_L = 16


def _pos_body(row_hbm, col_hbm, out_hbm, row_v, slab_v, sem, sem_c, sem_r):
    i = lax.axis_index("s") * _NC + lax.axis_index("c")
    c_col = pltpu.async_copy(col_hbm.at[pl.ds(0, _W)],
                             slab_v.at[:, pl.ds(0, _D)], sem_c)
    c_row = pltpu.async_copy(row_hbm.at[pl.ds(0, _H)], row_v, sem_r)
    c_row.wait()
    lanes = lax.iota(jnp.int32, _L)
    ii = jnp.full((_L,), i, jnp.int32)
    for k in range(_D // _L):
        v = plsc.load_gather(row_v, [ii, lanes + k * _L])
        for j in range(_W):
            slab_v[j, pl.ds(_D + k * _L, _L)] = v
    c_col.wait()
    copies = [pltpu.async_copy(slab_v, out_hbm.at[b, i], sem)
              for b in range(_B)]
    for cp in copies:
        cp.wait()


@jax.jit
def _pos_embed(row_embed, col_embed):
    mesh = plsc.VectorSubcoreMesh(core_axis_name="c", subcore_axis_name="s")
    out = pl.kernel(
        _pos_body,
        out_type=jax.ShapeDtypeStruct((_B, _H, _W, _C), jnp.float32),
        mesh=mesh,
        scratch_types=[
            pltpu.VMEM((_H, _D), jnp.float32),
            pltpu.VMEM((_W, _C), jnp.float32),
            pltpu.SemaphoreType.DMA,
            pltpu.SemaphoreType.DMA,
            pltpu.SemaphoreType.DMA,
        ],
        compiler_params=pltpu.CompilerParams(needs_layout_passes=False),
    )(row_embed, col_embed)
    return jnp.transpose(out, (0, 3, 1, 2))


def kernel(x, row_embed, col_embed):
    assert x.shape[0] == _B and x.shape[-2:] == (_H, _W)
    return _pos_embed(row_embed, col_embed)
